# baseline (speedup 1.0000x reference)
import functools

import jax
import jax.numpy as jnp
from jax import lax
from jax.experimental import pallas as pl
from jax.experimental.pallas import tpu as pltpu
from jax.experimental.pallas import tpu_sc as plsc

H = 128
N = 10000
E = 320000
G = 64

B = 64
NBLK = E // B
NC = 2
NS = 16
NW = NC * NS
ROW_CHUNK = 624
LAST_CHUNK = N - 15 * ROW_CHUNK
BPT = 157
NBLKP = BPT * NW
IDX_PAD = NBLKP * B - E


def _f32(x):
    return x.astype(jnp.float32)


def _node_lin_body(x_ref, w_ref, b_ref, o_ref):
    t = jnp.dot(x_ref[...], w_ref[...], preferred_element_type=jnp.float32)
    t = t + b_ref[...]
    o_ref[...] = t * jax.nn.sigmoid(t)


def _node_lin(x, wT, b2):
    return pl.pallas_call(
        _node_lin_body,
        out_shape=jax.ShapeDtypeStruct((N, H), jnp.float32),
    )(x, wT, b2)


BE = 2000


def _edge_mlp_body(ea_ref, eg1_ref, eg2_ref,
                   wa1, wb1, b11, w21, b21,
                   wa2, wb2, b12, w22, b22,
                   o1_ref, o2_ref):
    a = ea_ref[...]

    def branch(eg, wa, wb, b1, w2, b2):
        t = jnp.dot(a, wa[...], preferred_element_type=jnp.float32)
        t = t + jnp.dot(eg, wb[...], preferred_element_type=jnp.float32)
        t = t + b1[...]
        t = t * jax.nn.sigmoid(t)
        return jnp.dot(t, w2[...], preferred_element_type=jnp.float32) + b2[...]

    o1_ref[...] = branch(eg1_ref[...], wa1, wb1, b11, w21, b21)
    o2_ref[...] = branch(eg2_ref[...], wa2, wb2, b12, w22, b22)


def _edge_mlp(ea, eg1, eg2, ws):
    edge_spec = pl.BlockSpec((BE, H), lambda i: (i, 0))
    w_spec = pl.BlockSpec((H, H), lambda i: (0, 0))
    b_spec = pl.BlockSpec((1, H), lambda i: (0, 0))
    return pl.pallas_call(
        _edge_mlp_body,
        grid=(E // BE,),
        in_specs=[edge_spec, edge_spec, edge_spec,
                  w_spec, w_spec, b_spec, w_spec, b_spec,
                  w_spec, w_spec, b_spec, w_spec, b_spec],
        out_specs=[edge_spec, edge_spec],
        out_shape=[jax.ShapeDtypeStruct((E, H), jnp.float32),
                   jax.ShapeDtypeStruct((E, H), jnp.float32)],
    )(ea, eg1, eg2, *ws)


def _conv_sc(ew, x1, src2, dst2, wa, ba16, zrows):
    mesh = plsc.VectorSubcoreMesh(core_axis_name="c", subcore_axis_name="s")

    @functools.partial(
        pl.kernel,
        out_type=jax.ShapeDtypeStruct((NC, N, H), jnp.float32),
        mesh=mesh,
        scratch_types=[
            pltpu.VMEM_SHARED((N, H), jnp.float32),
            pltpu.VMEM((2, B), jnp.int32),
            pltpu.VMEM((2, B), jnp.int32),
            pltpu.VMEM((2, B, H), jnp.float32),
            pltpu.VMEM((2, B, H), jnp.float32),
            pltpu.VMEM((B, H), jnp.float32),
            pltpu.VMEM((H,), jnp.float32),
            pltpu.VMEM((16,), jnp.float32),
            pltpu.SemaphoreType.DMA,
            pltpu.SemaphoreType.DMA,
            pltpu.SemaphoreType.DMA,
            pltpu.SemaphoreType.DMA,
            pltpu.SemaphoreType.DMA,
            pltpu.SemaphoreType.DMA,
            pltpu.SemaphoreType.DMA,
            pltpu.SemaphoreType.DMA,
        ],
    )
    def k(ew_h, x1_h, src_h, dst_h, wa_h, ba_h, z_h, out_h,
          agg, siA, diA, ewv, xjv, ov, wav, bav,
          se0, se1, sg0, sg1, ss0, ss1, sd0, sd1):
        cid = lax.axis_index("c")
        sid = lax.axis_index("s")
        wid = sid * NC + cid
        r0 = sid * ROW_CHUNK

        @pl.when(sid < NS - 1)
        def _():
            pltpu.sync_copy(z_h.at[pl.ds(0, ROW_CHUNK)],
                            agg.at[pl.ds(r0, ROW_CHUNK)])

        @pl.when(sid == NS - 1)
        def _():
            pltpu.sync_copy(z_h, agg.at[pl.ds((NS - 1) * ROW_CHUNK, LAST_CHUNK)])

        start = wid * BPT
        pltpu.sync_copy(wa_h, wav)
        pltpu.sync_copy(ba_h, bav)
        plsc.subcore_barrier()

        nit = jnp.clip(NBLK - start, 0, BPT)
        wregs = [wav[pl.ds(kk * 16, 16)] for kk in range(8)]
        bareg = bav[...]
        lane = lax.iota(jnp.int32, 16)
        ses = (se0, se1)
        sgs = (sg0, sg1)
        sss = (ss0, ss1)
        sds = (sd0, sd1)

        def ibase(j):
            return (start + j) * B

        pltpu.sync_copy(src_h.at[pl.ds(ibase(0), B)], siA.at[0])
        pltpu.sync_copy(src_h.at[pl.ds(ibase(1), B)], siA.at[1])
        pltpu.sync_copy(dst_h.at[pl.ds(ibase(0), B)], diA.at[0])
        pltpu.sync_copy(dst_h.at[pl.ds(ibase(1), B)], diA.at[1])
        pltpu.async_copy(ew_h.at[pl.ds(ibase(0), B), :], ewv.at[0], se0)
        pltpu.async_copy(x1_h.at[siA.at[0]], xjv.at[0], sg0)

        def pair(i, carry):
            for b in range(2):
                j = i * 2 + b
                nb = 1 - b

                @pl.when(j < nit)
                def _(j=j, b=b, nb=nb):
                    @pl.when(j + 2 < nit)
                    def _():
                        pltpu.async_copy(src_h.at[pl.ds(ibase(j + 2), B)],
                                         siA.at[b], sss[b])

                    @pl.when(j + 1 < nit)
                    def _():
                        @pl.when(j >= 1)
                        def _():
                            pltpu.make_async_copy(
                                src_h.at[pl.ds(ibase(j + 1), B)],
                                siA.at[nb], sss[nb]).wait()
                        pltpu.async_copy(ew_h.at[pl.ds(ibase(j + 1), B), :],
                                         ewv.at[nb], ses[nb])
                        pltpu.async_copy(x1_h.at[siA.at[nb]], xjv.at[nb],
                                         sgs[nb])

                    pltpu.make_async_copy(
                        ew_h.at[pl.ds(0, B), :], ewv.at[b], ses[b]).wait()
                    pltpu.make_async_copy(
                        x1_h.at[siA.at[b]], xjv.at[b], sgs[b]).wait()

                    @plsc.parallel_loop(0, B, step=1, unroll=4)
                    def edge(e):
                        acc0 = bareg
                        acc1 = jnp.zeros((16,), jnp.float32)
                        ms = []
                        for kk in range(8):
                            mk = (ewv[b, e, pl.ds(kk * 16, 16)]
                                  * xjv[b, e, pl.ds(kk * 16, 16)])
                            if kk % 2 == 0:
                                acc0 = acc0 + mk * wregs[kk]
                            else:
                                acc1 = acc1 + mk * wregs[kk]
                            ms.append(mk)
                        acc = acc0 + acc1
                        for shift in (1, 2, 4, 8):
                            acc = acc + acc.at[lane ^ shift].get(
                                mode='promise_in_bounds')
                        attn = 1.0 / (1.0 + jnp.exp(-acc))
                        for kk in range(8):
                            ov[e, pl.ds(kk * 16, 16)] = ms[kk] * attn

                    @pl.when(j >= 2)
                    def _():
                        pltpu.make_async_copy(
                            dst_h.at[pl.ds(ibase(j), B)],
                            diA.at[b], sds[b]).wait()
                    pltpu.sync_copy(ov, agg.at[diA.at[b]], add=True)

                    @pl.when(j + 2 < nit)
                    def _():
                        pltpu.async_copy(dst_h.at[pl.ds(ibase(j + 2), B)],
                                         diA.at[b], sds[b])
            return carry

        lax.fori_loop(0, (BPT + 1) // 2, pair, 0)
        plsc.subcore_barrier()

        @pl.when(sid < NS - 1)
        def _():
            pltpu.sync_copy(agg.at[pl.ds(r0, ROW_CHUNK)],
                            out_h.at[cid, pl.ds(r0, ROW_CHUNK)])

        @pl.when(sid == NS - 1)
        def _():
            pltpu.sync_copy(agg.at[pl.ds((NS - 1) * ROW_CHUNK, LAST_CHUNK)],
                            out_h.at[cid, pl.ds((NS - 1) * ROW_CHUNK, LAST_CHUNK)])

    return k(ew, x1, src2, dst2, wa, ba16, zrows)


def _post_body(p1_ref, p2_ref, x1_ref, b_ref,
               rel1T, rel1b, root1T, lin1T, lin1b,
               rel2T, rel2b, root2T, lin2T, lin2b,
               c1T, c2T, catb, l0T, l0b, l1T, l1b,
               nw, nb, nms, finT, finb, o_ref):
    x1 = x1_ref[...]

    def head(p, relT, relb, rootT, linT, linb):
        agg = p[0] + p[1]
        o1 = jnp.dot(agg, relT[...], preferred_element_type=jnp.float32)
        o1 = o1 + relb[...]
        o1 = o1 + jnp.dot(x1, rootT[...], preferred_element_type=jnp.float32)
        t = jnp.dot(o1, linT[...], preferred_element_type=jnp.float32) + linb[...]
        return t * jax.nn.sigmoid(t)

    h1 = head(p1_ref[...], rel1T, rel1b, root1T, lin1T, lin1b)
    h2 = head(p2_ref[...], rel2T, rel2b, root2T, lin2T, lin2b)
    h = jnp.dot(h1, c1T[...], preferred_element_type=jnp.float32)
    h = h + jnp.dot(h2, c2T[...], preferred_element_type=jnp.float32)
    h = h + catb[...] + x1

    for wT, bb in ((l0T, l0b), (l1T, l1b)):
        t = jnp.dot(h, wT[...], preferred_element_type=jnp.float32) + bb[...]
        t = t * jax.nn.sigmoid(t) + h
        h = t * jax.nn.sigmoid(t) + t

    bvec = b_ref[...]
    onehot = (bvec == lax.broadcasted_iota(jnp.int32, (N, G), 1)).astype(jnp.float32)
    cnt = jnp.maximum(jnp.sum(onehot, axis=0), 1.0)
    sums = lax.dot_general(onehot, h, (((0,), (0,)), ((), ())),
                           preferred_element_type=jnp.float32)
    mean = sums / cnt[:, None]
    out = h - jnp.dot(onehot, mean, preferred_element_type=jnp.float32) * nms[...]
    var = lax.dot_general(onehot, out * out, (((0,), (0,)), ((), ())),
                          preferred_element_type=jnp.float32) / cnt[:, None]
    std = jnp.sqrt(var + 1e-5)
    hn = nw[...] * out / jnp.dot(onehot, std, preferred_element_type=jnp.float32)
    hn = hn + nb[...]
    o_ref[...] = jnp.dot(hn, finT[...], preferred_element_type=jnp.float32) + finb[...]


def _post(p1, p2, x1, batch2d, ws):
    return pl.pallas_call(
        _post_body,
        out_shape=jax.ShapeDtypeStruct((N, H), jnp.float32),
    )(p1, p2, x1, batch2d, *ws)


def kernel(x, edge_index, edge_attr, edge_geom_attr1, edge_geom_attr2, batch, params):
    p = params
    x1 = _node_lin(_f32(x), p['lin_W'].T, p['lin_b'][None])

    edge_ws = []
    for c in ('c1', 'c2'):
        el1 = p[c + '_el1_W']
        edge_ws.extend([el1[:, :H].T, el1[:, H:].T, p[c + '_el1_b'][None],
                        p[c + '_el2_W'].T, p[c + '_el2_b'][None]])
    ew1, ew2 = _edge_mlp(_f32(edge_attr), _f32(edge_geom_attr1),
                         _f32(edge_geom_attr2), edge_ws)

    src2 = jnp.pad(edge_index[0], (0, IDX_PAD))
    dst2 = jnp.pad(edge_index[1], (0, IDX_PAD))
    zrows = jnp.zeros((LAST_CHUNK, H), jnp.float32)
    parts = []
    for c, ew in (('c1', ew1), ('c2', ew2)):
        wa = p[c + '_ea_W'][0]
        ba16 = jnp.broadcast_to(p[c + '_ea_b'] * (1.0 / 16.0), (16,)).astype(jnp.float32)
        parts.append(_conv_sc(ew, x1, src2, dst2, wa, ba16, zrows))

    lincat = p['lincat_W']
    post_ws = [
        p['c1_rel_W'].T, p['c1_rel_b'][None], p['c1_root_W'].T,
        p['lin1_W'].T, p['lin1_b'][None],
        p['c2_rel_W'].T, p['c2_rel_b'][None], p['c2_root_W'].T,
        p['lin2_W'].T, p['lin2_b'][None],
        lincat[:, :H].T, lincat[:, H:].T, p['lincat_b'][None],
        p['l0_W'].T, p['l0_b'][None], p['l1_W'].T, p['l1_b'][None],
        p['norm_weight'][None], p['norm_bias'][None], p['norm_mean_scale'][None],
        p['final_W'].T, p['final_b'][None],
    ]
    return _post(parts[0], parts[1], x1, batch[:, None], post_ws)


# speedup vs baseline: 1.0406x; 1.0406x over previous
import functools

import jax
import jax.numpy as jnp
from jax import lax
from jax.experimental import pallas as pl
from jax.experimental.pallas import tpu as pltpu
from jax.experimental.pallas import tpu_sc as plsc

H = 128
N = 10000
E = 320000
G = 64

B = 64
NBLK = E // B
NC = 2
NS = 16
NW = NC * NS
ROW_CHUNK = 624
LAST_CHUNK = N - 15 * ROW_CHUNK
BPT = 157
NBLKP = BPT * NW
IDX_PAD = NBLKP * B - E


def _f32(x):
    return x.astype(jnp.float32)


def _node_lin_body(x_ref, w_ref, b_ref, o_ref):
    t = jnp.dot(x_ref[...], w_ref[...], preferred_element_type=jnp.float32)
    t = t + b_ref[...]
    o_ref[...] = t * jax.nn.sigmoid(t)


def _node_lin(x, wT, b2):
    return pl.pallas_call(
        _node_lin_body,
        out_shape=jax.ShapeDtypeStruct((N, H), jnp.float32),
    )(x, wT, b2)


BE = 2000


def _edge_mlp_body(ea_ref, eg1_ref, eg2_ref,
                   wa1, wb1, b11, w21, b21,
                   wa2, wb2, b12, w22, b22,
                   o1_ref, o2_ref):
    a = ea_ref[...]

    def branch(eg, wa, wb, b1, w2, b2):
        t = jnp.dot(a, wa[...], preferred_element_type=jnp.float32)
        t = t + jnp.dot(eg, wb[...], preferred_element_type=jnp.float32)
        t = t + b1[...]
        t = t * jax.nn.sigmoid(t)
        return jnp.dot(t, w2[...], preferred_element_type=jnp.float32) + b2[...]

    o1_ref[...] = branch(eg1_ref[...], wa1, wb1, b11, w21, b21)
    o2_ref[...] = branch(eg2_ref[...], wa2, wb2, b12, w22, b22)


def _edge_mlp(ea, eg1, eg2, ws):
    edge_spec = pl.BlockSpec((BE, H), lambda i: (i, 0))
    w_spec = pl.BlockSpec((H, H), lambda i: (0, 0))
    b_spec = pl.BlockSpec((1, H), lambda i: (0, 0))
    return pl.pallas_call(
        _edge_mlp_body,
        grid=(E // BE,),
        in_specs=[edge_spec, edge_spec, edge_spec,
                  w_spec, w_spec, b_spec, w_spec, b_spec,
                  w_spec, w_spec, b_spec, w_spec, b_spec],
        out_specs=[edge_spec, edge_spec],
        out_shape=[jax.ShapeDtypeStruct((E, H), jnp.float32),
                   jax.ShapeDtypeStruct((E, H), jnp.float32)],
    )(ea, eg1, eg2, *ws)


def _conv_sc(ew, x1, src2, dst2, wa, ba16, zrows):
    mesh = plsc.VectorSubcoreMesh(core_axis_name="c", subcore_axis_name="s")

    @functools.partial(
        pl.kernel,
        out_type=jax.ShapeDtypeStruct((NC, N, H), jnp.float32),
        mesh=mesh,
        scratch_types=[
            pltpu.VMEM_SHARED((N, H), jnp.float32),
            pltpu.VMEM((2, B), jnp.int32),
            pltpu.VMEM((2, B), jnp.int32),
            pltpu.VMEM((2, B, H), jnp.float32),
            pltpu.VMEM((2, B, H), jnp.float32),
            pltpu.VMEM((B, H), jnp.float32),
            pltpu.VMEM((H,), jnp.float32),
            pltpu.VMEM((16,), jnp.float32),
            pltpu.SemaphoreType.DMA,
            pltpu.SemaphoreType.DMA,
            pltpu.SemaphoreType.DMA,
            pltpu.SemaphoreType.DMA,
            pltpu.SemaphoreType.DMA,
            pltpu.SemaphoreType.DMA,
            pltpu.SemaphoreType.DMA,
            pltpu.SemaphoreType.DMA,
        ],
    )
    def k(ew_h, x1_h, src_h, dst_h, wa_h, ba_h, z_h, out_h,
          agg, siA, diA, ewv, xjv, ov, wav, bav,
          se0, se1, sg0, sg1, ss0, ss1, sd0, sd1):
        cid = lax.axis_index("c")
        sid = lax.axis_index("s")
        wid = sid * NC + cid
        r0 = sid * ROW_CHUNK

        @pl.when(sid < NS - 1)
        def _():
            pltpu.sync_copy(z_h.at[pl.ds(0, ROW_CHUNK)],
                            agg.at[pl.ds(r0, ROW_CHUNK)])

        @pl.when(sid == NS - 1)
        def _():
            pltpu.sync_copy(z_h, agg.at[pl.ds((NS - 1) * ROW_CHUNK, LAST_CHUNK)])

        start = wid * BPT
        pltpu.sync_copy(wa_h, wav)
        pltpu.sync_copy(ba_h, bav)
        plsc.subcore_barrier()

        nit = jnp.clip(NBLK - start, 0, BPT)
        wregs = [wav[pl.ds(kk * 16, 16)] for kk in range(8)]
        bareg = bav[...]
        lane = lax.iota(jnp.int32, 16)
        ses = (se0, se1)
        sgs = (sg0, sg1)
        sss = (ss0, ss1)
        sds = (sd0, sd1)

        def ibase(j):
            return (start + j) * B

        pltpu.sync_copy(src_h.at[pl.ds(ibase(0), B)], siA.at[0])
        pltpu.sync_copy(src_h.at[pl.ds(ibase(1), B)], siA.at[1])
        pltpu.sync_copy(dst_h.at[pl.ds(ibase(0), B)], diA.at[0])
        pltpu.sync_copy(dst_h.at[pl.ds(ibase(1), B)], diA.at[1])
        pltpu.async_copy(ew_h.at[pl.ds(ibase(0), B), :], ewv.at[0], se0)
        pltpu.async_copy(x1_h.at[siA.at[0]], xjv.at[0], sg0)

        def pair(i, carry):
            for b in range(2):
                j = i * 2 + b
                nb = 1 - b

                @pl.when(j < nit)
                def _(j=j, b=b, nb=nb):
                    @pl.when(j + 2 < nit)
                    def _():
                        pltpu.async_copy(src_h.at[pl.ds(ibase(j + 2), B)],
                                         siA.at[b], sss[b])

                    @pl.when(j + 1 < nit)
                    def _():
                        @pl.when(j >= 1)
                        def _():
                            pltpu.make_async_copy(
                                src_h.at[pl.ds(ibase(j + 1), B)],
                                siA.at[nb], sss[nb]).wait()
                        pltpu.async_copy(ew_h.at[pl.ds(ibase(j + 1), B), :],
                                         ewv.at[nb], ses[nb])
                        pltpu.async_copy(x1_h.at[siA.at[nb]], xjv.at[nb],
                                         sgs[nb])

                    pltpu.make_async_copy(
                        ew_h.at[pl.ds(0, B), :], ewv.at[b], ses[b]).wait()
                    pltpu.make_async_copy(
                        x1_h.at[siA.at[b]], xjv.at[b], sgs[b]).wait()

                    @plsc.parallel_loop(0, B, step=1, unroll=2)
                    def edge(e):
                        acc0 = bareg
                        acc1 = jnp.zeros((16,), jnp.float32)
                        ms = []
                        for kk in range(8):
                            mk = (ewv[b, e, pl.ds(kk * 16, 16)]
                                  * xjv[b, e, pl.ds(kk * 16, 16)])
                            if kk % 2 == 0:
                                acc0 = acc0 + mk * wregs[kk]
                            else:
                                acc1 = acc1 + mk * wregs[kk]
                            ms.append(mk)
                        acc = acc0 + acc1
                        for shift in (1, 2, 4, 8):
                            acc = acc + acc.at[lane ^ shift].get(
                                mode='promise_in_bounds')
                        attn = 1.0 / (1.0 + jnp.exp(-acc))
                        for kk in range(8):
                            ov[e, pl.ds(kk * 16, 16)] = ms[kk] * attn

                    @pl.when(j >= 2)
                    def _():
                        pltpu.make_async_copy(
                            dst_h.at[pl.ds(ibase(j), B)],
                            diA.at[b], sds[b]).wait()
                    pltpu.sync_copy(ov, agg.at[diA.at[b]], add=True)

                    @pl.when(j + 2 < nit)
                    def _():
                        pltpu.async_copy(dst_h.at[pl.ds(ibase(j + 2), B)],
                                         diA.at[b], sds[b])
            return carry

        lax.fori_loop(0, (BPT + 1) // 2, pair, 0)
        plsc.subcore_barrier()

        @pl.when(sid < NS - 1)
        def _():
            pltpu.sync_copy(agg.at[pl.ds(r0, ROW_CHUNK)],
                            out_h.at[cid, pl.ds(r0, ROW_CHUNK)])

        @pl.when(sid == NS - 1)
        def _():
            pltpu.sync_copy(agg.at[pl.ds((NS - 1) * ROW_CHUNK, LAST_CHUNK)],
                            out_h.at[cid, pl.ds((NS - 1) * ROW_CHUNK, LAST_CHUNK)])

    return k(ew, x1, src2, dst2, wa, ba16, zrows)


def _post_body(p1_ref, p2_ref, x1_ref, b_ref,
               rel1T, rel1b, root1T, lin1T, lin1b,
               rel2T, rel2b, root2T, lin2T, lin2b,
               c1T, c2T, catb, l0T, l0b, l1T, l1b,
               nw, nb, nms, finT, finb, o_ref):
    x1 = x1_ref[...]

    def head(p, relT, relb, rootT, linT, linb):
        agg = p[0] + p[1]
        o1 = jnp.dot(agg, relT[...], preferred_element_type=jnp.float32)
        o1 = o1 + relb[...]
        o1 = o1 + jnp.dot(x1, rootT[...], preferred_element_type=jnp.float32)
        t = jnp.dot(o1, linT[...], preferred_element_type=jnp.float32) + linb[...]
        return t * jax.nn.sigmoid(t)

    h1 = head(p1_ref[...], rel1T, rel1b, root1T, lin1T, lin1b)
    h2 = head(p2_ref[...], rel2T, rel2b, root2T, lin2T, lin2b)
    h = jnp.dot(h1, c1T[...], preferred_element_type=jnp.float32)
    h = h + jnp.dot(h2, c2T[...], preferred_element_type=jnp.float32)
    h = h + catb[...] + x1

    for wT, bb in ((l0T, l0b), (l1T, l1b)):
        t = jnp.dot(h, wT[...], preferred_element_type=jnp.float32) + bb[...]
        t = t * jax.nn.sigmoid(t) + h
        h = t * jax.nn.sigmoid(t) + t

    bvec = b_ref[...]
    onehot = (bvec == lax.broadcasted_iota(jnp.int32, (N, G), 1)).astype(jnp.float32)
    cnt = jnp.maximum(jnp.sum(onehot, axis=0), 1.0)
    sums = lax.dot_general(onehot, h, (((0,), (0,)), ((), ())),
                           preferred_element_type=jnp.float32)
    mean = sums / cnt[:, None]
    out = h - jnp.dot(onehot, mean, preferred_element_type=jnp.float32) * nms[...]
    var = lax.dot_general(onehot, out * out, (((0,), (0,)), ((), ())),
                          preferred_element_type=jnp.float32) / cnt[:, None]
    std = jnp.sqrt(var + 1e-5)
    hn = nw[...] * out / jnp.dot(onehot, std, preferred_element_type=jnp.float32)
    hn = hn + nb[...]
    o_ref[...] = jnp.dot(hn, finT[...], preferred_element_type=jnp.float32) + finb[...]


def _post(p1, p2, x1, batch2d, ws):
    return pl.pallas_call(
        _post_body,
        out_shape=jax.ShapeDtypeStruct((N, H), jnp.float32),
    )(p1, p2, x1, batch2d, *ws)


def kernel(x, edge_index, edge_attr, edge_geom_attr1, edge_geom_attr2, batch, params):
    p = params
    x1 = _node_lin(_f32(x), p['lin_W'].T, p['lin_b'][None])

    edge_ws = []
    for c in ('c1', 'c2'):
        el1 = p[c + '_el1_W']
        edge_ws.extend([el1[:, :H].T, el1[:, H:].T, p[c + '_el1_b'][None],
                        p[c + '_el2_W'].T, p[c + '_el2_b'][None]])
    ew1, ew2 = _edge_mlp(_f32(edge_attr), _f32(edge_geom_attr1),
                         _f32(edge_geom_attr2), edge_ws)

    src2 = jnp.pad(edge_index[0], (0, IDX_PAD))
    dst2 = jnp.pad(edge_index[1], (0, IDX_PAD))
    zrows = jnp.zeros((LAST_CHUNK, H), jnp.float32)
    parts = []
    for c, ew in (('c1', ew1), ('c2', ew2)):
        wa = p[c + '_ea_W'][0]
        ba16 = jnp.broadcast_to(p[c + '_ea_b'] * (1.0 / 16.0), (16,)).astype(jnp.float32)
        parts.append(_conv_sc(ew, x1, src2, dst2, wa, ba16, zrows))

    lincat = p['lincat_W']
    post_ws = [
        p['c1_rel_W'].T, p['c1_rel_b'][None], p['c1_root_W'].T,
        p['lin1_W'].T, p['lin1_b'][None],
        p['c2_rel_W'].T, p['c2_rel_b'][None], p['c2_root_W'].T,
        p['lin2_W'].T, p['lin2_b'][None],
        lincat[:, :H].T, lincat[:, H:].T, p['lincat_b'][None],
        p['l0_W'].T, p['l0_b'][None], p['l1_W'].T, p['l1_b'][None],
        p['norm_weight'][None], p['norm_bias'][None], p['norm_mean_scale'][None],
        p['final_W'].T, p['final_b'][None],
    ]
    return _post(parts[0], parts[1], x1, batch[:, None], post_ws)


# speedup vs baseline: 1.1049x; 1.0618x over previous
import functools

import jax
import jax.numpy as jnp
from jax import lax
from jax.experimental import pallas as pl
from jax.experimental.pallas import tpu as pltpu
from jax.experimental.pallas import tpu_sc as plsc

H = 128
N = 10000
E = 320000
G = 64

B = 64
NBLK = E // B
NC = 2
NS = 16
NW = NC * NS
ROW_CHUNK = 624
LAST_CHUNK = N - 15 * ROW_CHUNK
BPT = 157
NBLKP = BPT * NW
IDX_PAD = NBLKP * B - E


def _f32(x):
    return x.astype(jnp.float32)


def _node_lin_body(x_ref, w_ref, b_ref, o_ref):
    t = jnp.dot(x_ref[...], w_ref[...], preferred_element_type=jnp.float32)
    t = t + b_ref[...]
    o_ref[...] = t * jax.nn.sigmoid(t)


def _node_lin(x, wT, b2):
    return pl.pallas_call(
        _node_lin_body,
        out_shape=jax.ShapeDtypeStruct((N, H), jnp.float32),
    )(x, wT, b2)


BE = 2000


def _edge_mlp_body(ea_ref, eg_ref, wa, wb, b1, w2, b2, o_ref):
    t = jnp.dot(ea_ref[...], wa[...], preferred_element_type=jnp.float32)
    t = t + jnp.dot(eg_ref[...], wb[...], preferred_element_type=jnp.float32)
    t = t + b1[...]
    t = t * jax.nn.sigmoid(t)
    o_ref[...] = jnp.dot(t, w2[...], preferred_element_type=jnp.float32) + b2[...]


def _edge_mlp(ea, eg, ws):
    edge_spec = pl.BlockSpec((BE, H), lambda i: (i, 0))
    w_spec = pl.BlockSpec((H, H), lambda i: (0, 0))
    b_spec = pl.BlockSpec((1, H), lambda i: (0, 0))
    return pl.pallas_call(
        _edge_mlp_body,
        grid=(E // BE,),
        in_specs=[edge_spec, edge_spec,
                  w_spec, w_spec, b_spec, w_spec, b_spec],
        out_specs=edge_spec,
        out_shape=jax.ShapeDtypeStruct((E, H), jnp.float32),
    )(ea, eg, *ws)


def _conv_sc(ew, x1, src2, dst2, wa, ba16, zrows):
    mesh = plsc.VectorSubcoreMesh(core_axis_name="c", subcore_axis_name="s")

    @functools.partial(
        pl.kernel,
        out_type=jax.ShapeDtypeStruct((NC, N, H), jnp.float32),
        mesh=mesh,
        scratch_types=[
            pltpu.VMEM_SHARED((N, H), jnp.float32),
            pltpu.VMEM((2, B), jnp.int32),
            pltpu.VMEM((2, B), jnp.int32),
            pltpu.VMEM((2, B, H), jnp.float32),
            pltpu.VMEM((2, B, H), jnp.float32),
            pltpu.VMEM((B, H), jnp.float32),
            pltpu.VMEM((H,), jnp.float32),
            pltpu.VMEM((16,), jnp.float32),
            pltpu.SemaphoreType.DMA,
            pltpu.SemaphoreType.DMA,
            pltpu.SemaphoreType.DMA,
            pltpu.SemaphoreType.DMA,
            pltpu.SemaphoreType.DMA,
            pltpu.SemaphoreType.DMA,
            pltpu.SemaphoreType.DMA,
            pltpu.SemaphoreType.DMA,
        ],
    )
    def k(ew_h, x1_h, src_h, dst_h, wa_h, ba_h, z_h, out_h,
          agg, siA, diA, ewv, xjv, ov, wav, bav,
          se0, se1, sg0, sg1, ss0, ss1, sd0, sd1):
        cid = lax.axis_index("c")
        sid = lax.axis_index("s")
        wid = sid * NC + cid
        r0 = sid * ROW_CHUNK

        @pl.when(sid < NS - 1)
        def _():
            pltpu.sync_copy(z_h.at[pl.ds(0, ROW_CHUNK)],
                            agg.at[pl.ds(r0, ROW_CHUNK)])

        @pl.when(sid == NS - 1)
        def _():
            pltpu.sync_copy(z_h, agg.at[pl.ds((NS - 1) * ROW_CHUNK, LAST_CHUNK)])

        start = wid * BPT
        pltpu.sync_copy(wa_h, wav)
        pltpu.sync_copy(ba_h, bav)
        plsc.subcore_barrier()

        nit = jnp.clip(NBLK - start, 0, BPT)
        wregs = [wav[pl.ds(kk * 16, 16)] for kk in range(8)]
        bareg = bav[...]
        lane = lax.iota(jnp.int32, 16)
        ses = (se0, se1)
        sgs = (sg0, sg1)
        sss = (ss0, ss1)
        sds = (sd0, sd1)

        def ibase(j):
            return (start + j) * B

        pltpu.sync_copy(src_h.at[pl.ds(ibase(0), B)], siA.at[0])
        pltpu.sync_copy(src_h.at[pl.ds(ibase(1), B)], siA.at[1])
        pltpu.sync_copy(dst_h.at[pl.ds(ibase(0), B)], diA.at[0])
        pltpu.sync_copy(dst_h.at[pl.ds(ibase(1), B)], diA.at[1])
        pltpu.async_copy(ew_h.at[pl.ds(ibase(0), B), :], ewv.at[0], se0)
        pltpu.async_copy(x1_h.at[siA.at[0]], xjv.at[0], sg0)

        def pair(i, carry):
            for b in range(2):
                j = i * 2 + b
                nb = 1 - b

                @pl.when(j < nit)
                def _(j=j, b=b, nb=nb):
                    @pl.when(j + 2 < nit)
                    def _():
                        pltpu.async_copy(src_h.at[pl.ds(ibase(j + 2), B)],
                                         siA.at[b], sss[b])

                    @pl.when(j + 1 < nit)
                    def _():
                        @pl.when(j >= 1)
                        def _():
                            pltpu.make_async_copy(
                                src_h.at[pl.ds(ibase(j + 1), B)],
                                siA.at[nb], sss[nb]).wait()
                        pltpu.async_copy(ew_h.at[pl.ds(ibase(j + 1), B), :],
                                         ewv.at[nb], ses[nb])
                        pltpu.async_copy(x1_h.at[siA.at[nb]], xjv.at[nb],
                                         sgs[nb])

                    pltpu.make_async_copy(
                        ew_h.at[pl.ds(0, B), :], ewv.at[b], ses[b]).wait()
                    pltpu.make_async_copy(
                        x1_h.at[siA.at[b]], xjv.at[b], sgs[b]).wait()

                    def edge(e, c2):
                        acc0 = bareg
                        acc1 = jnp.zeros((16,), jnp.float32)
                        ms = []
                        for kk in range(8):
                            mk = (ewv[b, e, pl.ds(kk * 16, 16)]
                                  * xjv[b, e, pl.ds(kk * 16, 16)])
                            if kk % 2 == 0:
                                acc0 = acc0 + mk * wregs[kk]
                            else:
                                acc1 = acc1 + mk * wregs[kk]
                            ms.append(mk)
                        acc = acc0 + acc1
                        for shift in (1, 2, 4, 8):
                            acc = acc + acc.at[lane ^ shift].get(
                                mode='promise_in_bounds')
                        attn = 1.0 / (1.0 + jnp.exp(-acc))
                        for kk in range(8):
                            ov[e, pl.ds(kk * 16, 16)] = ms[kk] * attn
                        return c2

                    lax.fori_loop(0, B, edge, 0)

                    @pl.when(j >= 2)
                    def _():
                        pltpu.make_async_copy(
                            dst_h.at[pl.ds(ibase(j), B)],
                            diA.at[b], sds[b]).wait()
                    pltpu.sync_copy(ov, agg.at[diA.at[b]], add=True)

                    @pl.when(j + 2 < nit)
                    def _():
                        pltpu.async_copy(dst_h.at[pl.ds(ibase(j + 2), B)],
                                         diA.at[b], sds[b])
            return carry

        lax.fori_loop(0, (BPT + 1) // 2, pair, 0)
        plsc.subcore_barrier()

        @pl.when(sid < NS - 1)
        def _():
            pltpu.sync_copy(agg.at[pl.ds(r0, ROW_CHUNK)],
                            out_h.at[cid, pl.ds(r0, ROW_CHUNK)])

        @pl.when(sid == NS - 1)
        def _():
            pltpu.sync_copy(agg.at[pl.ds((NS - 1) * ROW_CHUNK, LAST_CHUNK)],
                            out_h.at[cid, pl.ds((NS - 1) * ROW_CHUNK, LAST_CHUNK)])

    return k(ew, x1, src2, dst2, wa, ba16, zrows)


def _post_body(p1_ref, p2_ref, x1_ref, b_ref,
               rel1T, rel1b, root1T, lin1T, lin1b,
               rel2T, rel2b, root2T, lin2T, lin2b,
               c1T, c2T, catb, l0T, l0b, l1T, l1b,
               nw, nb, nms, finT, finb, o_ref):
    x1 = x1_ref[...]

    def head(p, relT, relb, rootT, linT, linb):
        agg = p[0] + p[1]
        o1 = jnp.dot(agg, relT[...], preferred_element_type=jnp.float32)
        o1 = o1 + relb[...]
        o1 = o1 + jnp.dot(x1, rootT[...], preferred_element_type=jnp.float32)
        t = jnp.dot(o1, linT[...], preferred_element_type=jnp.float32) + linb[...]
        return t * jax.nn.sigmoid(t)

    h1 = head(p1_ref[...], rel1T, rel1b, root1T, lin1T, lin1b)
    h2 = head(p2_ref[...], rel2T, rel2b, root2T, lin2T, lin2b)
    h = jnp.dot(h1, c1T[...], preferred_element_type=jnp.float32)
    h = h + jnp.dot(h2, c2T[...], preferred_element_type=jnp.float32)
    h = h + catb[...] + x1

    for wT, bb in ((l0T, l0b), (l1T, l1b)):
        t = jnp.dot(h, wT[...], preferred_element_type=jnp.float32) + bb[...]
        t = t * jax.nn.sigmoid(t) + h
        h = t * jax.nn.sigmoid(t) + t

    bvec = b_ref[...]
    onehot = (bvec == lax.broadcasted_iota(jnp.int32, (N, G), 1)).astype(jnp.float32)
    cnt = jnp.maximum(jnp.sum(onehot, axis=0), 1.0)
    sums = lax.dot_general(onehot, h, (((0,), (0,)), ((), ())),
                           preferred_element_type=jnp.float32)
    mean = sums / cnt[:, None]
    out = h - jnp.dot(onehot, mean, preferred_element_type=jnp.float32) * nms[...]
    var = lax.dot_general(onehot, out * out, (((0,), (0,)), ((), ())),
                          preferred_element_type=jnp.float32) / cnt[:, None]
    std = jnp.sqrt(var + 1e-5)
    hn = nw[...] * out / jnp.dot(onehot, std, preferred_element_type=jnp.float32)
    hn = hn + nb[...]
    o_ref[...] = jnp.dot(hn, finT[...], preferred_element_type=jnp.float32) + finb[...]


def _post(p1, p2, x1, batch2d, ws):
    return pl.pallas_call(
        _post_body,
        out_shape=jax.ShapeDtypeStruct((N, H), jnp.float32),
    )(p1, p2, x1, batch2d, *ws)


def kernel(x, edge_index, edge_attr, edge_geom_attr1, edge_geom_attr2, batch, params):
    p = params
    x1 = _node_lin(_f32(x), p['lin_W'].T, p['lin_b'][None])

    src2 = jnp.pad(edge_index[0], (0, IDX_PAD))
    dst2 = jnp.pad(edge_index[1], (0, IDX_PAD))
    zrows = jnp.zeros((LAST_CHUNK, H), jnp.float32)
    parts = []
    for c, eg in (('c1', edge_geom_attr1), ('c2', edge_geom_attr2)):
        el1 = p[c + '_el1_W']
        ew = _edge_mlp(_f32(edge_attr), _f32(eg),
                       [el1[:, :H].T, el1[:, H:].T, p[c + '_el1_b'][None],
                        p[c + '_el2_W'].T, p[c + '_el2_b'][None]])
        wa = p[c + '_ea_W'][0]
        ba16 = jnp.broadcast_to(p[c + '_ea_b'] * (1.0 / 16.0), (16,)).astype(jnp.float32)
        parts.append(_conv_sc(ew, x1, src2, dst2, wa, ba16, zrows))

    lincat = p['lincat_W']
    post_ws = [
        p['c1_rel_W'].T, p['c1_rel_b'][None], p['c1_root_W'].T,
        p['lin1_W'].T, p['lin1_b'][None],
        p['c2_rel_W'].T, p['c2_rel_b'][None], p['c2_root_W'].T,
        p['lin2_W'].T, p['lin2_b'][None],
        lincat[:, :H].T, lincat[:, H:].T, p['lincat_b'][None],
        p['l0_W'].T, p['l0_b'][None], p['l1_W'].T, p['l1_b'][None],
        p['norm_weight'][None], p['norm_bias'][None], p['norm_mean_scale'][None],
        p['final_W'].T, p['final_b'][None],
    ]
    return _post(parts[0], parts[1], x1, batch[:, None], post_ws)


# speedup vs baseline: 1.1577x; 1.0477x over previous
import functools

import jax
import jax.numpy as jnp
from jax import lax
from jax.experimental import pallas as pl
from jax.experimental.pallas import tpu as pltpu
from jax.experimental.pallas import tpu_sc as plsc

H = 128
N = 10000
E = 320000
G = 64

B = 64
NBLK = E // B
NCHUNK = 2
CBLK = NBLK // NCHUNK
NC = 2
NS = 16
NW = NC * NS
BPT = -(-CBLK // NW)
ROW_CHUNK = 624
LAST_CHUNK = N - 15 * ROW_CHUNK


def _f32(x):
    return x.astype(jnp.float32)


def _node_lin_body(x_ref, w_ref, b_ref, o_ref):
    t = jnp.dot(x_ref[...], w_ref[...], preferred_element_type=jnp.float32)
    t = t + b_ref[...]
    o_ref[...] = t * jax.nn.sigmoid(t)


def _node_lin(x, wT, b2):
    return pl.pallas_call(
        _node_lin_body,
        out_shape=jax.ShapeDtypeStruct((N, H), jnp.float32),
    )(x, wT, b2)


BE = 2000


def _edge_mlp_body(ea_ref, eg_ref, wa, wb, b1, w2, b2, o_ref):
    t = jnp.dot(ea_ref[...], wa[...], preferred_element_type=jnp.float32)
    t = t + jnp.dot(eg_ref[...], wb[...], preferred_element_type=jnp.float32)
    t = t + b1[...]
    t = t * jax.nn.sigmoid(t)
    o_ref[...] = jnp.dot(t, w2[...], preferred_element_type=jnp.float32) + b2[...]


EC = E // NCHUNK


def _edge_mlp(ea, eg, ws, ch):
    off = ch * (EC // BE)
    in_spec = pl.BlockSpec((BE, H), lambda i: (i + off, 0))
    out_spec = pl.BlockSpec((BE, H), lambda i: (i, 0))
    w_spec = pl.BlockSpec((H, H), lambda i: (0, 0))
    b_spec = pl.BlockSpec((1, H), lambda i: (0, 0))
    return pl.pallas_call(
        _edge_mlp_body,
        grid=(EC // BE,),
        in_specs=[in_spec, in_spec,
                  w_spec, w_spec, b_spec, w_spec, b_spec],
        out_specs=out_spec,
        out_shape=jax.ShapeDtypeStruct((EC, H), jnp.float32),
    )(ea, eg, *ws)


def _conv_sc(ew, x1, src, dst, wa, ba16, zrows, cbase):
    mesh = plsc.VectorSubcoreMesh(core_axis_name="c", subcore_axis_name="s")

    @functools.partial(
        pl.kernel,
        out_type=jax.ShapeDtypeStruct((NC, N, H), jnp.float32),
        mesh=mesh,
        scratch_types=[
            pltpu.VMEM_SHARED((N, H), jnp.float32),
            pltpu.VMEM((2, B), jnp.int32),
            pltpu.VMEM((2, B), jnp.int32),
            pltpu.VMEM((2, B, H), jnp.float32),
            pltpu.VMEM((2, B, H), jnp.float32),
            pltpu.VMEM((B, H), jnp.float32),
            pltpu.VMEM((H,), jnp.float32),
            pltpu.VMEM((16,), jnp.float32),
            pltpu.SemaphoreType.DMA,
            pltpu.SemaphoreType.DMA,
            pltpu.SemaphoreType.DMA,
            pltpu.SemaphoreType.DMA,
            pltpu.SemaphoreType.DMA,
            pltpu.SemaphoreType.DMA,
            pltpu.SemaphoreType.DMA,
            pltpu.SemaphoreType.DMA,
        ],
    )
    def k(ew_h, x1_h, src_h, dst_h, wa_h, ba_h, z_h, out_h,
          agg, siA, diA, ewv, xjv, ov, wav, bav,
          se0, se1, sg0, sg1, ss0, ss1, sd0, sd1):
        cid = lax.axis_index("c")
        sid = lax.axis_index("s")
        wid = sid * NC + cid
        r0 = sid * ROW_CHUNK

        @pl.when(sid < NS - 1)
        def _():
            pltpu.sync_copy(z_h.at[pl.ds(0, ROW_CHUNK)],
                            agg.at[pl.ds(r0, ROW_CHUNK)])

        @pl.when(sid == NS - 1)
        def _():
            pltpu.sync_copy(z_h, agg.at[pl.ds((NS - 1) * ROW_CHUNK, LAST_CHUNK)])

        start = wid * BPT
        pltpu.sync_copy(wa_h, wav)
        pltpu.sync_copy(ba_h, bav)
        plsc.subcore_barrier()

        nit = jnp.clip(CBLK - start, 0, BPT)
        wregs = [wav[pl.ds(kk * 16, 16)] for kk in range(8)]
        bareg = bav[...]
        lane = lax.iota(jnp.int32, 16)
        ses = (se0, se1)
        sgs = (sg0, sg1)
        sss = (ss0, ss1)
        sds = (sd0, sd1)

        def ebase(j):
            return (start + j) * B

        def ibase(j):
            return (cbase + start + j) * B

        pltpu.sync_copy(src_h.at[pl.ds(ibase(0), B)], siA.at[0])
        pltpu.sync_copy(src_h.at[pl.ds(ibase(1), B)], siA.at[1])
        pltpu.sync_copy(dst_h.at[pl.ds(ibase(0), B)], diA.at[0])
        pltpu.sync_copy(dst_h.at[pl.ds(ibase(1), B)], diA.at[1])
        pltpu.async_copy(ew_h.at[pl.ds(ebase(0), B), :], ewv.at[0], se0)
        pltpu.async_copy(x1_h.at[siA.at[0]], xjv.at[0], sg0)

        def pair(i, carry):
            for b in range(2):
                j = i * 2 + b
                nb = 1 - b

                @pl.when(j < nit)
                def _(j=j, b=b, nb=nb):
                    @pl.when(j + 2 < nit)
                    def _():
                        pltpu.async_copy(src_h.at[pl.ds(ibase(j + 2), B)],
                                         siA.at[b], sss[b])

                    @pl.when(j + 1 < nit)
                    def _():
                        @pl.when(j >= 1)
                        def _():
                            pltpu.make_async_copy(
                                src_h.at[pl.ds(ibase(j + 1), B)],
                                siA.at[nb], sss[nb]).wait()
                        pltpu.async_copy(ew_h.at[pl.ds(ebase(j + 1), B), :],
                                         ewv.at[nb], ses[nb])
                        pltpu.async_copy(x1_h.at[siA.at[nb]], xjv.at[nb],
                                         sgs[nb])

                    pltpu.make_async_copy(
                        ew_h.at[pl.ds(0, B), :], ewv.at[b], ses[b]).wait()
                    pltpu.make_async_copy(
                        x1_h.at[siA.at[b]], xjv.at[b], sgs[b]).wait()

                    def edge(e, c2):
                        acc0 = bareg
                        acc1 = jnp.zeros((16,), jnp.float32)
                        ms = []
                        for kk in range(8):
                            mk = (ewv[b, e, pl.ds(kk * 16, 16)]
                                  * xjv[b, e, pl.ds(kk * 16, 16)])
                            if kk % 2 == 0:
                                acc0 = acc0 + mk * wregs[kk]
                            else:
                                acc1 = acc1 + mk * wregs[kk]
                            ms.append(mk)
                        acc = acc0 + acc1
                        for shift in (1, 2, 4, 8):
                            acc = acc + acc.at[lane ^ shift].get(
                                mode='promise_in_bounds')
                        attn = 1.0 / (1.0 + jnp.exp(-acc))
                        for kk in range(8):
                            ov[e, pl.ds(kk * 16, 16)] = ms[kk] * attn
                        return c2

                    lax.fori_loop(0, B, edge, 0)

                    @pl.when(j >= 2)
                    def _():
                        pltpu.make_async_copy(
                            dst_h.at[pl.ds(ibase(j), B)],
                            diA.at[b], sds[b]).wait()
                    pltpu.sync_copy(ov, agg.at[diA.at[b]], add=True)

                    @pl.when(j + 2 < nit)
                    def _():
                        pltpu.async_copy(dst_h.at[pl.ds(ibase(j + 2), B)],
                                         diA.at[b], sds[b])
            return carry

        lax.fori_loop(0, (BPT + 1) // 2, pair, 0)
        plsc.subcore_barrier()

        @pl.when(sid < NS - 1)
        def _():
            pltpu.sync_copy(agg.at[pl.ds(r0, ROW_CHUNK)],
                            out_h.at[cid, pl.ds(r0, ROW_CHUNK)])

        @pl.when(sid == NS - 1)
        def _():
            pltpu.sync_copy(agg.at[pl.ds((NS - 1) * ROW_CHUNK, LAST_CHUNK)],
                            out_h.at[cid, pl.ds((NS - 1) * ROW_CHUNK, LAST_CHUNK)])

    return k(ew, x1, src, dst, wa, ba16, zrows)


BS = 1000


def _sum_parts_body(pa_ref, pb_ref, o_ref):
    o_ref[...] = (pa_ref[0] + pa_ref[1]) + (pb_ref[0] + pb_ref[1])


def _sum_parts(pa, pb):
    part_spec = pl.BlockSpec((NC, BS, H), lambda i: (0, i, 0))
    return pl.pallas_call(
        _sum_parts_body,
        grid=(N // BS,),
        in_specs=[part_spec, part_spec],
        out_specs=pl.BlockSpec((BS, H), lambda i: (i, 0)),
        out_shape=jax.ShapeDtypeStruct((N, H), jnp.float32),
    )(pa, pb)


def _post_body(agg1_ref, agg2_ref, x1_ref, b_ref,
               rel1T, rel1b, root1T, lin1T, lin1b,
               rel2T, rel2b, root2T, lin2T, lin2b,
               c1T, c2T, catb, l0T, l0b, l1T, l1b,
               nw, nb, nms, finT, finb, o_ref):
    x1 = x1_ref[...]

    def head(agg, relT, relb, rootT, linT, linb):
        o1 = jnp.dot(agg, relT[...], preferred_element_type=jnp.float32)
        o1 = o1 + relb[...]
        o1 = o1 + jnp.dot(x1, rootT[...], preferred_element_type=jnp.float32)
        t = jnp.dot(o1, linT[...], preferred_element_type=jnp.float32) + linb[...]
        return t * jax.nn.sigmoid(t)

    h1 = head(agg1_ref[...], rel1T, rel1b, root1T, lin1T, lin1b)
    h2 = head(agg2_ref[...], rel2T, rel2b, root2T, lin2T, lin2b)
    h = jnp.dot(h1, c1T[...], preferred_element_type=jnp.float32)
    h = h + jnp.dot(h2, c2T[...], preferred_element_type=jnp.float32)
    h = h + catb[...] + x1

    for wT, bb in ((l0T, l0b), (l1T, l1b)):
        t = jnp.dot(h, wT[...], preferred_element_type=jnp.float32) + bb[...]
        t = t * jax.nn.sigmoid(t) + h
        h = t * jax.nn.sigmoid(t) + t

    bvec = b_ref[...]
    onehot = (bvec == lax.broadcasted_iota(jnp.int32, (N, G), 1)).astype(jnp.float32)
    cnt = jnp.maximum(jnp.sum(onehot, axis=0), 1.0)
    sums = lax.dot_general(onehot, h, (((0,), (0,)), ((), ())),
                           preferred_element_type=jnp.float32)
    mean = sums / cnt[:, None]
    out = h - jnp.dot(onehot, mean, preferred_element_type=jnp.float32) * nms[...]
    var = lax.dot_general(onehot, out * out, (((0,), (0,)), ((), ())),
                          preferred_element_type=jnp.float32) / cnt[:, None]
    std = jnp.sqrt(var + 1e-5)
    hn = nw[...] * out / jnp.dot(onehot, std, preferred_element_type=jnp.float32)
    hn = hn + nb[...]
    o_ref[...] = jnp.dot(hn, finT[...], preferred_element_type=jnp.float32) + finb[...]


def _post(agg1, agg2, x1, batch2d, ws):
    return pl.pallas_call(
        _post_body,
        out_shape=jax.ShapeDtypeStruct((N, H), jnp.float32),
    )(agg1, agg2, x1, batch2d, *ws)


def kernel(x, edge_index, edge_attr, edge_geom_attr1, edge_geom_attr2, batch, params):
    p = params
    x1 = _node_lin(_f32(x), p['lin_W'].T, p['lin_b'][None])

    src = edge_index[0]
    dst = edge_index[1]
    zrows = jnp.zeros((LAST_CHUNK, H), jnp.float32)
    parts = []
    for c, eg in (('c1', edge_geom_attr1), ('c2', edge_geom_attr2)):
        el1 = p[c + '_el1_W']
        wa = p[c + '_ea_W'][0]
        ba16 = jnp.broadcast_to(p[c + '_ea_b'] * (1.0 / 16.0), (16,)).astype(jnp.float32)
        ews = [el1[:, :H].T, el1[:, H:].T, p[c + '_el1_b'][None],
               p[c + '_el2_W'].T, p[c + '_el2_b'][None]]
        for ch in range(NCHUNK):
            ew = _edge_mlp(_f32(edge_attr), _f32(eg), ews, ch)
            parts.append(_conv_sc(ew, x1, src, dst, wa, ba16, zrows,
                                  ch * CBLK))

    lincat = p['lincat_W']
    post_ws = [
        p['c1_rel_W'].T, p['c1_rel_b'][None], p['c1_root_W'].T,
        p['lin1_W'].T, p['lin1_b'][None],
        p['c2_rel_W'].T, p['c2_rel_b'][None], p['c2_root_W'].T,
        p['lin2_W'].T, p['lin2_b'][None],
        lincat[:, :H].T, lincat[:, H:].T, p['lincat_b'][None],
        p['l0_W'].T, p['l0_b'][None], p['l1_W'].T, p['l1_b'][None],
        p['norm_weight'][None], p['norm_bias'][None], p['norm_mean_scale'][None],
        p['final_W'].T, p['final_b'][None],
    ]
    agg1 = _sum_parts(parts[0], parts[1])
    agg2 = _sum_parts(parts[2], parts[3])
    return _post(agg1, agg2, x1, batch[:, None], post_ws)


# speedup vs baseline: 1.1627x; 1.0044x over previous
import functools

import jax
import jax.numpy as jnp
from jax import lax
from jax.experimental import pallas as pl
from jax.experimental.pallas import tpu as pltpu
from jax.experimental.pallas import tpu_sc as plsc

H = 128
N = 10000
E = 320000
G = 64

B = 64
NBLK = E // B
NCHUNK = 2
CBLK = NBLK // NCHUNK
NC = 2
NS = 16
NW = NC * NS
BPT = -(-CBLK // NW)
ROW_CHUNK = 624
LAST_CHUNK = N - 15 * ROW_CHUNK


def _f32(x):
    return x.astype(jnp.float32)


def _node_lin_body(x_ref, w_ref, b_ref, o_ref):
    t = jnp.dot(x_ref[...], w_ref[...], preferred_element_type=jnp.float32)
    t = t + b_ref[...]
    o_ref[...] = t * jax.nn.sigmoid(t)


def _node_lin(x, wT, b2):
    return pl.pallas_call(
        _node_lin_body,
        out_shape=jax.ShapeDtypeStruct((N, H), jnp.float32),
    )(x, wT, b2)


BE = 2000


def _edge_mlp_body(ea_ref, eg_ref, wa, wb, b1, w2, b2, o_ref):
    t = jnp.dot(ea_ref[...], wa[...], preferred_element_type=jnp.float32)
    t = t + jnp.dot(eg_ref[...], wb[...], preferred_element_type=jnp.float32)
    t = t + b1[...]
    t = t * jax.nn.sigmoid(t)
    o_ref[...] = jnp.dot(t, w2[...], preferred_element_type=jnp.float32) + b2[...]


EC = E // NCHUNK


def _edge_mlp(ea, eg, ws, ch):
    off = ch * (EC // BE)
    in_spec = pl.BlockSpec((BE, H), lambda i: (i + off, 0))
    out_spec = pl.BlockSpec((BE, H), lambda i: (i, 0))
    w_spec = pl.BlockSpec((H, H), lambda i: (0, 0))
    b_spec = pl.BlockSpec((1, H), lambda i: (0, 0))
    return pl.pallas_call(
        _edge_mlp_body,
        grid=(EC // BE,),
        in_specs=[in_spec, in_spec,
                  w_spec, w_spec, b_spec, w_spec, b_spec],
        out_specs=out_spec,
        out_shape=jax.ShapeDtypeStruct((EC, H), jnp.float32),
    )(ea, eg, *ws)


def _conv_sc(ew, x1, src, dst, wa, ba16, zrows, cbase):
    mesh = plsc.VectorSubcoreMesh(core_axis_name="c", subcore_axis_name="s")

    @functools.partial(
        pl.kernel,
        out_type=jax.ShapeDtypeStruct((NC, N, H), jnp.float32),
        mesh=mesh,
        scratch_types=[
            pltpu.VMEM_SHARED((N, H), jnp.float32),
            pltpu.VMEM((2, B), jnp.int32),
            pltpu.VMEM((2, B), jnp.int32),
            pltpu.VMEM((2, B, H), jnp.float32),
            pltpu.VMEM((2, B, H), jnp.float32),
            pltpu.VMEM((2, B, H), jnp.float32),
            pltpu.VMEM((H,), jnp.float32),
            pltpu.VMEM((16,), jnp.float32),
            pltpu.SemaphoreType.DMA,
            pltpu.SemaphoreType.DMA,
            pltpu.SemaphoreType.DMA,
            pltpu.SemaphoreType.DMA,
            pltpu.SemaphoreType.DMA,
            pltpu.SemaphoreType.DMA,
            pltpu.SemaphoreType.DMA,
            pltpu.SemaphoreType.DMA,
            pltpu.SemaphoreType.DMA,
            pltpu.SemaphoreType.DMA,
        ],
    )
    def k(ew_h, x1_h, src_h, dst_h, wa_h, ba_h, z_h, out_h,
          agg, siA, diA, ewv, xjv, ov, wav, bav,
          se0, se1, sg0, sg1, ss0, ss1, sd0, sd1, sc0, sc1):
        cid = lax.axis_index("c")
        sid = lax.axis_index("s")
        wid = sid * NC + cid
        r0 = sid * ROW_CHUNK

        @pl.when(sid < NS - 1)
        def _():
            pltpu.sync_copy(z_h.at[pl.ds(0, ROW_CHUNK)],
                            agg.at[pl.ds(r0, ROW_CHUNK)])

        @pl.when(sid == NS - 1)
        def _():
            pltpu.sync_copy(z_h, agg.at[pl.ds((NS - 1) * ROW_CHUNK, LAST_CHUNK)])

        start = wid * BPT
        pltpu.sync_copy(wa_h, wav)
        pltpu.sync_copy(ba_h, bav)
        plsc.subcore_barrier()

        nit = jnp.clip(CBLK - start, 0, BPT)
        wregs = [wav[pl.ds(kk * 16, 16)] for kk in range(8)]
        bareg = bav[...]
        lane = lax.iota(jnp.int32, 16)
        ses = (se0, se1)
        sgs = (sg0, sg1)
        sss = (ss0, ss1)
        sds = (sd0, sd1)
        scs = (sc0, sc1)

        def ebase(j):
            return (start + j) * B

        def ibase(j):
            return (cbase + start + j) * B

        pltpu.sync_copy(src_h.at[pl.ds(ibase(0), B)], siA.at[0])
        pltpu.sync_copy(src_h.at[pl.ds(ibase(1), B)], siA.at[1])
        pltpu.async_copy(ew_h.at[pl.ds(ebase(0), B), :], ewv.at[0], se0)
        pltpu.async_copy(x1_h.at[siA.at[0]], xjv.at[0], sg0)

        def pair(i, carry):
            for b in range(2):
                j = i * 2 + b
                nb = 1 - b

                @pl.when(j < nit)
                def _(j=j, b=b, nb=nb):
                    @pl.when(j + 2 < nit)
                    def _():
                        pltpu.async_copy(src_h.at[pl.ds(ibase(j + 2), B)],
                                         siA.at[b], sss[b])

                    @pl.when(j + 1 < nit)
                    def _():
                        @pl.when(j >= 1)
                        def _():
                            pltpu.make_async_copy(
                                src_h.at[pl.ds(ibase(j + 1), B)],
                                siA.at[nb], sss[nb]).wait()
                        pltpu.async_copy(ew_h.at[pl.ds(ebase(j + 1), B), :],
                                         ewv.at[nb], ses[nb])
                        pltpu.async_copy(x1_h.at[siA.at[nb]], xjv.at[nb],
                                         sgs[nb])

                    pltpu.make_async_copy(
                        ew_h.at[pl.ds(0, B), :], ewv.at[b], ses[b]).wait()
                    pltpu.make_async_copy(
                        x1_h.at[siA.at[b]], xjv.at[b], sgs[b]).wait()

                    @pl.when(j >= 2)
                    def _():
                        pltpu.make_async_copy(
                            ov.at[b], agg.at[diA.at[b]], scs[b]).wait()
                    pltpu.async_copy(dst_h.at[pl.ds(ibase(j), B)],
                                     diA.at[b], sds[b])

                    def edge(e, c2):
                        acc0 = bareg
                        acc1 = jnp.zeros((16,), jnp.float32)
                        ms = []
                        for kk in range(8):
                            mk = (ewv[b, e, pl.ds(kk * 16, 16)]
                                  * xjv[b, e, pl.ds(kk * 16, 16)])
                            if kk % 2 == 0:
                                acc0 = acc0 + mk * wregs[kk]
                            else:
                                acc1 = acc1 + mk * wregs[kk]
                            ms.append(mk)
                        acc = acc0 + acc1
                        for shift in (1, 2, 4, 8):
                            acc = acc + acc.at[lane ^ shift].get(
                                mode='promise_in_bounds')
                        attn = 1.0 / (1.0 + jnp.exp(-acc))
                        for kk in range(8):
                            ov[b, e, pl.ds(kk * 16, 16)] = ms[kk] * attn
                        return c2

                    lax.fori_loop(0, B, edge, 0)

                    pltpu.make_async_copy(
                        dst_h.at[pl.ds(ibase(j), B)],
                        diA.at[b], sds[b]).wait()
                    pltpu.async_copy(ov.at[b], agg.at[diA.at[b]], scs[b],
                                     add=True)
            return carry

        lax.fori_loop(0, (BPT + 1) // 2, pair, 0)
        pltpu.make_async_copy(ov.at[0], agg.at[diA.at[0]], scs[0]).wait()
        pltpu.make_async_copy(ov.at[1], agg.at[diA.at[1]], scs[1]).wait()
        plsc.subcore_barrier()

        @pl.when(sid < NS - 1)
        def _():
            pltpu.sync_copy(agg.at[pl.ds(r0, ROW_CHUNK)],
                            out_h.at[cid, pl.ds(r0, ROW_CHUNK)])

        @pl.when(sid == NS - 1)
        def _():
            pltpu.sync_copy(agg.at[pl.ds((NS - 1) * ROW_CHUNK, LAST_CHUNK)],
                            out_h.at[cid, pl.ds((NS - 1) * ROW_CHUNK, LAST_CHUNK)])

    return k(ew, x1, src, dst, wa, ba16, zrows)


BS = 1000


def _sum_parts_body(pa_ref, pb_ref, o_ref):
    o_ref[...] = (pa_ref[0] + pa_ref[1]) + (pb_ref[0] + pb_ref[1])


def _sum_parts(pa, pb):
    part_spec = pl.BlockSpec((NC, BS, H), lambda i: (0, i, 0))
    return pl.pallas_call(
        _sum_parts_body,
        grid=(N // BS,),
        in_specs=[part_spec, part_spec],
        out_specs=pl.BlockSpec((BS, H), lambda i: (i, 0)),
        out_shape=jax.ShapeDtypeStruct((N, H), jnp.float32),
    )(pa, pb)


def _post_body(agg1_ref, agg2_ref, x1_ref, b_ref,
               rel1T, rel1b, root1T, lin1T, lin1b,
               rel2T, rel2b, root2T, lin2T, lin2b,
               c1T, c2T, catb, l0T, l0b, l1T, l1b,
               nw, nb, nms, finT, finb, o_ref):
    x1 = x1_ref[...]

    def head(agg, relT, relb, rootT, linT, linb):
        o1 = jnp.dot(agg, relT[...], preferred_element_type=jnp.float32)
        o1 = o1 + relb[...]
        o1 = o1 + jnp.dot(x1, rootT[...], preferred_element_type=jnp.float32)
        t = jnp.dot(o1, linT[...], preferred_element_type=jnp.float32) + linb[...]
        return t * jax.nn.sigmoid(t)

    h1 = head(agg1_ref[...], rel1T, rel1b, root1T, lin1T, lin1b)
    h2 = head(agg2_ref[...], rel2T, rel2b, root2T, lin2T, lin2b)
    h = jnp.dot(h1, c1T[...], preferred_element_type=jnp.float32)
    h = h + jnp.dot(h2, c2T[...], preferred_element_type=jnp.float32)
    h = h + catb[...] + x1

    for wT, bb in ((l0T, l0b), (l1T, l1b)):
        t = jnp.dot(h, wT[...], preferred_element_type=jnp.float32) + bb[...]
        t = t * jax.nn.sigmoid(t) + h
        h = t * jax.nn.sigmoid(t) + t

    bvec = b_ref[...]
    onehot = (bvec == lax.broadcasted_iota(jnp.int32, (N, G), 1)).astype(jnp.float32)
    cnt = jnp.maximum(jnp.sum(onehot, axis=0), 1.0)
    sums = lax.dot_general(onehot, h, (((0,), (0,)), ((), ())),
                           preferred_element_type=jnp.float32)
    mean = sums / cnt[:, None]
    out = h - jnp.dot(onehot, mean, preferred_element_type=jnp.float32) * nms[...]
    var = lax.dot_general(onehot, out * out, (((0,), (0,)), ((), ())),
                          preferred_element_type=jnp.float32) / cnt[:, None]
    std = jnp.sqrt(var + 1e-5)
    hn = nw[...] * out / jnp.dot(onehot, std, preferred_element_type=jnp.float32)
    hn = hn + nb[...]
    o_ref[...] = jnp.dot(hn, finT[...], preferred_element_type=jnp.float32) + finb[...]


def _post(agg1, agg2, x1, batch2d, ws):
    return pl.pallas_call(
        _post_body,
        out_shape=jax.ShapeDtypeStruct((N, H), jnp.float32),
    )(agg1, agg2, x1, batch2d, *ws)


def kernel(x, edge_index, edge_attr, edge_geom_attr1, edge_geom_attr2, batch, params):
    p = params
    x1 = _node_lin(_f32(x), p['lin_W'].T, p['lin_b'][None])

    src = edge_index[0]
    dst = edge_index[1]
    zrows = jnp.zeros((LAST_CHUNK, H), jnp.float32)
    parts = []
    for c, eg in (('c1', edge_geom_attr1), ('c2', edge_geom_attr2)):
        el1 = p[c + '_el1_W']
        wa = p[c + '_ea_W'][0]
        ba16 = jnp.broadcast_to(p[c + '_ea_b'] * (1.0 / 16.0), (16,)).astype(jnp.float32)
        ews = [el1[:, :H].T, el1[:, H:].T, p[c + '_el1_b'][None],
               p[c + '_el2_W'].T, p[c + '_el2_b'][None]]
        for ch in range(NCHUNK):
            ew = _edge_mlp(_f32(edge_attr), _f32(eg), ews, ch)
            parts.append(_conv_sc(ew, x1, src, dst, wa, ba16, zrows,
                                  ch * CBLK))

    lincat = p['lincat_W']
    post_ws = [
        p['c1_rel_W'].T, p['c1_rel_b'][None], p['c1_root_W'].T,
        p['lin1_W'].T, p['lin1_b'][None],
        p['c2_rel_W'].T, p['c2_rel_b'][None], p['c2_root_W'].T,
        p['lin2_W'].T, p['lin2_b'][None],
        lincat[:, :H].T, lincat[:, H:].T, p['lincat_b'][None],
        p['l0_W'].T, p['l0_b'][None], p['l1_W'].T, p['l1_b'][None],
        p['norm_weight'][None], p['norm_bias'][None], p['norm_mean_scale'][None],
        p['final_W'].T, p['final_b'][None],
    ]
    agg1 = _sum_parts(parts[0], parts[1])
    agg2 = _sum_parts(parts[2], parts[3])
    return _post(agg1, agg2, x1, batch[:, None], post_ws)
